# baseline (speedup 1.0000x reference)
import dataclasses
import functools

import jax
import jax.numpy as jnp
from jax import lax
from jax.experimental import pallas as pl
from jax.experimental.pallas import tpu as pltpu
from jax.experimental.pallas import tpu_sc as plsc

N = 10000
K = 5000
D = 256
S = 16384
ROWS, LANES = 8, 2048

NW = 32
RPW = 160
CH = 8
CPAD = 5008
G4 = 78
XC = 8
IDXPAD = 5120


def _topk_sort_kernel(scores_ref, kout_ref, vout_ref):
    k = scores_ref[...]
    r = lax.broadcasted_iota(jnp.int32, (ROWS, LANES), 0)
    c = lax.broadcasted_iota(jnp.int32, (ROWS, LANES), 1)
    flat = r * LANES + c
    v = flat

    def partner(x, j):
        if j < LANES:
            bit = (c & j) != 0
            up = pltpu.roll(x, LANES - j, 1)
            dn = pltpu.roll(x, j, 1)
            return jnp.where(bit, dn, up)
        m = j // LANES
        bit = (r & m) != 0
        up = jnp.concatenate([x[m:], x[:m]], axis=0)
        dn = jnp.concatenate([x[-m:], x[:-m]], axis=0)
        return jnp.where(bit, dn, up)

    kk = 2
    while kk <= S:
        j = kk // 2
        while j >= 1:
            kp = partner(k, j)
            vp = partner(v, j)
            ilow = (flat & j) == 0
            asc = (flat & kk) == 0
            before = (k > kp) | ((k == kp) & (v < vp))
            take_mine = before == (ilow == asc)
            k = jnp.where(take_mine, k, kp)
            v = jnp.where(take_mine, v, vp)
            j //= 2
        kk *= 2

    kout_ref[...] = k
    vout_ref[...] = v


_topk_call = pl.pallas_call(
    _topk_sort_kernel,
    out_shape=(
        jax.ShapeDtypeStruct((ROWS, LANES), jnp.float32),
        jax.ShapeDtypeStruct((ROWS, LANES), jnp.int32),
    ),
)


NSTRIPE = N // 128
TAILOFF = NSTRIPE * 128
ABUFW = TAILOFF + 128


SUB = 2
NSUB = RPW // SUB
NBUF = 4
DEPTH = 3


def _sc_gather_body(a_hbm, atail_hbm, x_hbm, idx_hbm, idxq_hbm, vals_hbm,
                    a2_hbm, nx_hbm, colidx_v, myidx_v, myidxq_v, myvals_v,
                    abufa_v, abufb_v, abufc_v, abufd_v, obuf_v, xbuf_v,
                    gsema, gsemb, gsemc, gsemd, osem, xgsem, xosem):
    cid = lax.axis_index("c")
    sid = lax.axis_index("s")
    w = sid * 2 + cid
    base = pl.multiple_of(w * RPW, RPW)

    pltpu.sync_copy(idx_hbm.at[pl.ds(0, CPAD)], colidx_v)
    pltpu.sync_copy(idx_hbm.at[pl.ds(base, RPW)], myidx_v)
    pltpu.sync_copy(
        idxq_hbm.at[pl.ds(pl.multiple_of(w * NSUB * 8, NSUB * 8), NSUB * 8)],
        myidxq_v)
    pltpu.sync_copy(vals_hbm.at[pl.ds(base, RPW)], myvals_v)

    lane = lax.iota(jnp.int32, 16)

    def _fire(s, buf, sem):
        rows_ref = myidxq_v.at[pl.ds(s * 8, SUB)]
        pltpu.make_async_copy(a_hbm.at[rows_ref, pl.ds(0, TAILOFF)],
                              buf.at[:, pl.ds(0, TAILOFF)], sem).start()
        pltpu.make_async_copy(atail_hbm.at[rows_ref],
                              buf.at[:, pl.ds(TAILOFF, 128)], sem).start()

    def _wait_gather(buf, sem):
        pltpu.make_async_copy(a_hbm.at[pl.ds(0, SUB), pl.ds(0, TAILOFF)],
                              buf.at[:, pl.ds(0, TAILOFF)], sem).wait()
        pltpu.make_async_copy(atail_hbm.at[pl.ds(0, SUB)],
                              buf.at[:, pl.ds(TAILOFF, 128)], sem).wait()

    def _drain_write():
        pltpu.make_async_copy(a2_hbm.at[pl.ds(0, CH)], obuf_v, osem).wait()

    def _compute(buf, half):
        for rr in range(SUB):
            rows16 = jnp.full((16,), rr, jnp.int32)
            orow = half * SUB + rr

            @plsc.parallel_loop(0, G4, 1, unroll=6)
            def _grp(g):
                for q in range(4):
                    off = (g * 4 + q) * 16
                    cols = colidx_v[pl.ds(off, 16)]
                    vals = plsc.load_gather(buf, [rows16, cols])
                    obuf_v[orow, pl.ds(off, 16)] = vals

            cols = colidx_v[pl.ds(G4 * 64, 16)]
            vals = plsc.load_gather(buf, [rows16, cols])
            plsc.store_scatter(
                obuf_v, [jnp.full((16,), orow, jnp.int32), lane + G4 * 64],
                vals, mask=lane < (K - G4 * 64))

    abufs = (abufa_v, abufb_v, abufc_v, abufd_v)
    gsems = (gsema, gsemb, gsemc, gsemd)

    for p in range(DEPTH):
        @pl.when(base + p * SUB < K)
        def _():
            _fire(p, abufs[p], gsems[p])

    @pl.loop(0, RPW // CH)
    def _a_chunk(tt):
        s0 = NBUF * tt
        row_even = base + s0 * SUB
        chunk_row0 = pl.multiple_of(base + tt * CH, CH)

        @pl.when(row_even < K)
        def _():
            @pl.when(tt > 0)
            def _():
                pltpu.make_async_copy(nx_hbm.at[pl.ds(0, XC)], xbuf_v,
                                      xosem).wait()
            pltpu.make_async_copy(x_hbm.at[myidx_v.at[pl.ds(tt * XC, XC)]],
                                  xbuf_v, xgsem).start()

        @pl.when((tt > 0) & (row_even < K))
        def _():
            _drain_write()

        for p in range(NBUF):
            s = s0 + p

            @pl.when(row_even < K)
            def _():
                _wait_gather(abufs[p], gsems[p])
                _compute(abufs[p], p)

            @pl.when((s + DEPTH < NSUB) & (base + (s + DEPTH) * SUB < K))
            def _():
                _fire(s + DEPTH, abufs[(p + DEPTH) % NBUF],
                      gsems[(p + DEPTH) % NBUF])

        @pl.when(row_even < K)
        def _():
            pltpu.make_async_copy(obuf_v, a2_hbm.at[pl.ds(chunk_row0, CH)],
                                  osem).start()

        @pl.when(row_even < K)
        def _():
            pltpu.make_async_copy(x_hbm.at[pl.ds(0, XC)], xbuf_v,
                                  xgsem).wait()
            for rr in range(XC):
                vv = plsc.load_gather(
                    myvals_v, [jnp.full((16,), tt * XC + rr, jnp.int32)])

                @pl.loop(0, D // 16)
                def _seg(gx):
                    xbuf_v[rr, pl.ds(gx * 16, 16)] = (
                        xbuf_v[rr, pl.ds(gx * 16, 16)] * vv)

            pltpu.make_async_copy(xbuf_v, nx_hbm.at[pl.ds(chunk_row0, XC)],
                                  xosem).start()

    @pl.when(base < K)
    def _():
        _drain_write()
        pltpu.make_async_copy(nx_hbm.at[pl.ds(0, XC)], xbuf_v, xosem).wait()


@functools.cache
def _sc_gather_call():
    mesh = plsc.VectorSubcoreMesh(
        core_axis_name="c", subcore_axis_name="s", num_cores=2, num_subcores=16
    )
    cp = pltpu.CompilerParams()
    if "needs_layout_passes" in pltpu.CompilerParams.__dataclass_fields__:
        cp = dataclasses.replace(cp, needs_layout_passes=False)
    return pl.kernel(
        _sc_gather_body,
        out_type=(
            jax.ShapeDtypeStruct((K, K), jnp.float32),
            jax.ShapeDtypeStruct((K, D), jnp.float32),
        ),
        mesh=mesh,
        compiler_params=cp,
        scratch_types=[
            pltpu.VMEM((CPAD,), jnp.int32),
            pltpu.VMEM((RPW,), jnp.int32),
            pltpu.VMEM((NSUB * 8,), jnp.int32),
            pltpu.VMEM((RPW,), jnp.float32),
            pltpu.VMEM((SUB, ABUFW), jnp.float32),
            pltpu.VMEM((SUB, ABUFW), jnp.float32),
            pltpu.VMEM((SUB, ABUFW), jnp.float32),
            pltpu.VMEM((SUB, ABUFW), jnp.float32),
            pltpu.VMEM((CH, K), jnp.float32),
            pltpu.VMEM((XC, D), jnp.float32),
            pltpu.SemaphoreType.DMA,
            pltpu.SemaphoreType.DMA,
            pltpu.SemaphoreType.DMA,
            pltpu.SemaphoreType.DMA,
            pltpu.SemaphoreType.DMA,
            pltpu.SemaphoreType.DMA,
            pltpu.SemaphoreType.DMA,
        ],
    )


def kernel(A, X, W, b):
    scores = X @ W.T + b
    scores = jnp.squeeze(scores, -1)
    scores = jax.nn.sigmoid(scores / 100.0)

    spad = jnp.pad(scores, (0, S - N), constant_values=-1.0)
    skeys, sidx = _topk_call(spad.reshape(ROWS, LANES))
    values = skeys.reshape(-1)[:K]
    idx = sidx.reshape(-1)[:K]

    idxp = jnp.pad(idx, (0, IDXPAD - K))
    valsp = jnp.pad(values, (0, IDXPAD - K))
    A_tail = jnp.pad(A[:, TAILOFF:], ((0, 0), (0, 128 - (N - TAILOFF))))
    idxq8 = jnp.pad(idxp.reshape(-1, SUB), ((0, 0), (0, 8 - SUB))).reshape(-1)
    A2, new_X = _sc_gather_call()(A, A_tail, X, idxp, idxq8, valsp)
    return (A2, new_X, idx)


# speedup vs baseline: 1.0177x; 1.0177x over previous
import dataclasses
import functools

import jax
import jax.numpy as jnp
from jax import lax
from jax.experimental import pallas as pl
from jax.experimental.pallas import tpu as pltpu
from jax.experimental.pallas import tpu_sc as plsc

N = 10000
K = 5000
D = 256
S = 16384
ROWS, LANES = 8, 2048

NW = 32
RPW = 160
CH = 8
CPAD = 5008
G4 = 78
XC = 8
IDXPAD = 5120


def _topk_sort_kernel(scores_ref, kout_ref, vout_ref):
    k = scores_ref[...]
    r = lax.broadcasted_iota(jnp.int32, (ROWS, LANES), 0)
    c = lax.broadcasted_iota(jnp.int32, (ROWS, LANES), 1)
    flat = r * LANES + c
    v = flat

    def partner(x, j):
        if j < LANES:
            bit = (c & j) != 0
            up = pltpu.roll(x, LANES - j, 1)
            dn = pltpu.roll(x, j, 1)
            return jnp.where(bit, dn, up)
        m = j // LANES
        bit = (r & m) != 0
        up = jnp.concatenate([x[m:], x[:m]], axis=0)
        dn = jnp.concatenate([x[-m:], x[:-m]], axis=0)
        return jnp.where(bit, dn, up)

    kk = 2
    while kk <= S:
        j = kk // 2
        while j >= 1:
            kp = partner(k, j)
            vp = partner(v, j)
            ilow = (flat & j) == 0
            asc = (flat & kk) == 0
            before = (k > kp) | ((k == kp) & (v < vp))
            take_mine = before == (ilow == asc)
            k = jnp.where(take_mine, k, kp)
            v = jnp.where(take_mine, v, vp)
            j //= 2
        kk *= 2

    kout_ref[...] = k
    vout_ref[...] = v


_topk_call = pl.pallas_call(
    _topk_sort_kernel,
    out_shape=(
        jax.ShapeDtypeStruct((ROWS, LANES), jnp.float32),
        jax.ShapeDtypeStruct((ROWS, LANES), jnp.int32),
    ),
)


NSTRIPE = N // 128
TAILOFF = NSTRIPE * 128
ABUFW = TAILOFF + 128


SUB = 2
NSUB = RPW // SUB
NBUF = 4
DEPTH = 3


def _sc_gather_body(a_hbm, atail_hbm, x_hbm, idx_hbm, idxq_hbm, vals_hbm,
                    a2_hbm, nx_hbm, colidx_v, myidx_v, myidxq_v, myvals_v,
                    abufa_v, abufb_v, abufc_v, abufd_v, obuf_v, xbuf_v,
                    gsema, gsemb, gsemc, gsemd, osem, xgsem, xosem):
    cid = lax.axis_index("c")
    sid = lax.axis_index("s")
    w = sid * 2 + cid
    base = pl.multiple_of(w * RPW, RPW)

    pltpu.sync_copy(idx_hbm.at[pl.ds(0, CPAD)], colidx_v)
    pltpu.sync_copy(idx_hbm.at[pl.ds(base, RPW)], myidx_v)
    pltpu.sync_copy(
        idxq_hbm.at[pl.ds(pl.multiple_of(w * NSUB * 8, NSUB * 8), NSUB * 8)],
        myidxq_v)
    pltpu.sync_copy(vals_hbm.at[pl.ds(base, RPW)], myvals_v)

    lane = lax.iota(jnp.int32, 16)

    def _fire(s, buf, sem):
        rows_ref = myidxq_v.at[pl.ds(s * 8, SUB)]
        pltpu.make_async_copy(a_hbm.at[rows_ref, pl.ds(0, TAILOFF)],
                              buf.at[:, pl.ds(0, TAILOFF)], sem).start()
        pltpu.make_async_copy(atail_hbm.at[rows_ref],
                              buf.at[:, pl.ds(TAILOFF, 128)], sem).start()

    def _wait_gather(buf, sem):
        pltpu.make_async_copy(a_hbm.at[pl.ds(0, SUB), pl.ds(0, TAILOFF)],
                              buf.at[:, pl.ds(0, TAILOFF)], sem).wait()
        pltpu.make_async_copy(atail_hbm.at[pl.ds(0, SUB)],
                              buf.at[:, pl.ds(TAILOFF, 128)], sem).wait()

    def _drain_write():
        pltpu.make_async_copy(a2_hbm.at[pl.ds(0, CH)], obuf_v, osem).wait()

    def _compute(buf, half):
        for rr in range(SUB):
            rows16 = jnp.full((16,), rr, jnp.int32)
            orow = half * SUB + rr

            @plsc.parallel_loop(0, G4, 1, unroll=2)
            def _grp(g):
                for q in range(4):
                    off = (g * 4 + q) * 16
                    cols = colidx_v[pl.ds(off, 16)]
                    vals = plsc.load_gather(buf, [rows16, cols])
                    obuf_v[orow, pl.ds(off, 16)] = vals

            cols = colidx_v[pl.ds(G4 * 64, 16)]
            vals = plsc.load_gather(buf, [rows16, cols])
            plsc.store_scatter(
                obuf_v, [jnp.full((16,), orow, jnp.int32), lane + G4 * 64],
                vals, mask=lane < (K - G4 * 64))

    abufs = (abufa_v, abufb_v, abufc_v, abufd_v)
    gsems = (gsema, gsemb, gsemc, gsemd)

    for p in range(DEPTH):
        @pl.when(base + p * SUB < K)
        def _():
            _fire(p, abufs[p], gsems[p])

    @pl.loop(0, RPW // CH)
    def _a_chunk(tt):
        s0 = NBUF * tt
        row_even = base + s0 * SUB
        chunk_row0 = pl.multiple_of(base + tt * CH, CH)

        @pl.when(row_even < K)
        def _():
            @pl.when(tt > 0)
            def _():
                pltpu.make_async_copy(nx_hbm.at[pl.ds(0, XC)], xbuf_v,
                                      xosem).wait()
            pltpu.make_async_copy(x_hbm.at[myidx_v.at[pl.ds(tt * XC, XC)]],
                                  xbuf_v, xgsem).start()

        @pl.when((tt > 0) & (row_even < K))
        def _():
            _drain_write()

        for p in range(NBUF):
            s = s0 + p

            @pl.when(row_even < K)
            def _():
                _wait_gather(abufs[p], gsems[p])
                _compute(abufs[p], p)

            @pl.when((s + DEPTH < NSUB) & (base + (s + DEPTH) * SUB < K))
            def _():
                _fire(s + DEPTH, abufs[(p + DEPTH) % NBUF],
                      gsems[(p + DEPTH) % NBUF])

        @pl.when(row_even < K)
        def _():
            pltpu.make_async_copy(obuf_v, a2_hbm.at[pl.ds(chunk_row0, CH)],
                                  osem).start()

        @pl.when(row_even < K)
        def _():
            pltpu.make_async_copy(x_hbm.at[pl.ds(0, XC)], xbuf_v,
                                  xgsem).wait()
            for rr in range(XC):
                vv = plsc.load_gather(
                    myvals_v, [jnp.full((16,), tt * XC + rr, jnp.int32)])

                @pl.loop(0, D // 16)
                def _seg(gx):
                    xbuf_v[rr, pl.ds(gx * 16, 16)] = (
                        xbuf_v[rr, pl.ds(gx * 16, 16)] * vv)

            pltpu.make_async_copy(xbuf_v, nx_hbm.at[pl.ds(chunk_row0, XC)],
                                  xosem).start()

    @pl.when(base < K)
    def _():
        _drain_write()
        pltpu.make_async_copy(nx_hbm.at[pl.ds(0, XC)], xbuf_v, xosem).wait()


@functools.cache
def _sc_gather_call():
    mesh = plsc.VectorSubcoreMesh(
        core_axis_name="c", subcore_axis_name="s", num_cores=2, num_subcores=16
    )
    cp = pltpu.CompilerParams()
    if "needs_layout_passes" in pltpu.CompilerParams.__dataclass_fields__:
        cp = dataclasses.replace(cp, needs_layout_passes=False)
    return pl.kernel(
        _sc_gather_body,
        out_type=(
            jax.ShapeDtypeStruct((K, K), jnp.float32),
            jax.ShapeDtypeStruct((K, D), jnp.float32),
        ),
        mesh=mesh,
        compiler_params=cp,
        scratch_types=[
            pltpu.VMEM((CPAD,), jnp.int32),
            pltpu.VMEM((RPW,), jnp.int32),
            pltpu.VMEM((NSUB * 8,), jnp.int32),
            pltpu.VMEM((RPW,), jnp.float32),
            pltpu.VMEM((SUB, ABUFW), jnp.float32),
            pltpu.VMEM((SUB, ABUFW), jnp.float32),
            pltpu.VMEM((SUB, ABUFW), jnp.float32),
            pltpu.VMEM((SUB, ABUFW), jnp.float32),
            pltpu.VMEM((CH, K), jnp.float32),
            pltpu.VMEM((XC, D), jnp.float32),
            pltpu.SemaphoreType.DMA,
            pltpu.SemaphoreType.DMA,
            pltpu.SemaphoreType.DMA,
            pltpu.SemaphoreType.DMA,
            pltpu.SemaphoreType.DMA,
            pltpu.SemaphoreType.DMA,
            pltpu.SemaphoreType.DMA,
        ],
    )


def kernel(A, X, W, b):
    scores = X @ W.T + b
    scores = jnp.squeeze(scores, -1)
    scores = jax.nn.sigmoid(scores / 100.0)

    spad = jnp.pad(scores, (0, S - N), constant_values=-1.0)
    skeys, sidx = _topk_call(spad.reshape(ROWS, LANES))
    values = skeys.reshape(-1)[:K]
    idx = sidx.reshape(-1)[:K]

    idxp = jnp.pad(idx, (0, IDXPAD - K))
    valsp = jnp.pad(values, (0, IDXPAD - K))
    A_tail = jnp.pad(A[:, TAILOFF:], ((0, 0), (0, 128 - (N - TAILOFF))))
    idxq8 = jnp.pad(idxp.reshape(-1, SUB), ((0, 0), (0, 8 - SUB))).reshape(-1)
    A2, new_X = _sc_gather_call()(A, A_tail, X, idxp, idxq8, valsp)
    return (A2, new_X, idx)
